# baseline (speedup 1.0000x reference)
import jax
import jax.numpy as jnp
from jax import lax
from jax.experimental import pallas as pl
from jax.experimental.pallas import tpu as pltpu
from jax.experimental.pallas import tpu_sc as plsc

_N = 10000
_NCORE = 2
_NSUB = 16
_CHUNK = 64
_NACC = 10000


def _tc1_body(x_ref, we_ref, wd_ref, hx_ref, ad_ref):
    xb = x_ref[...]
    hx_ref[...] = jnp.dot(xb, we_ref[...], preferred_element_type=jnp.float32)
    ad_ref[...] = jnp.dot(xb, wd_ref[...], preferred_element_type=jnp.float32)


def _tc1(x, wext, wdst):
    B = 1000
    return pl.pallas_call(
        _tc1_body,
        grid=(_N // B,),
        in_specs=[
            pl.BlockSpec((B, 128), lambda i: (i, 0)),
            pl.BlockSpec((128, 144), lambda i: (0, 0)),
            pl.BlockSpec((128, 16), lambda i: (0, 0)),
        ],
        out_specs=[
            pl.BlockSpec((B, 144), lambda i: (i, 0)),
            pl.BlockSpec((B, 16), lambda i: (i, 0)),
        ],
        out_shape=[
            jax.ShapeDtypeStruct((_N, 144), jnp.float32),
            jax.ShapeDtypeStruct((_N, 16), jnp.float32),
        ],
    )(x, wext, wdst)


def _tc2_body(p0_ref, p1_ref, b1_ref, rmat_ref, w2e_ref, w2d_ref, hx2_ref, ad2_ref):
    acc = p0_ref[...] + p1_ref[...]
    feats = acc[:, :128]
    den = acc[:, 128:136]
    denb = jnp.dot(den, rmat_ref[...], preferred_element_type=jnp.float32) + 1e-16
    z = feats / denb + b1_ref[...]
    z = jnp.where(z > 0.0, z, jnp.exp(jnp.minimum(z, 0.0)) - 1.0)
    hx2_ref[...] = jnp.dot(z, w2e_ref[...], preferred_element_type=jnp.float32)
    ad2_ref[...] = jnp.dot(z, w2d_ref[...], preferred_element_type=jnp.float32)


def _tc2(p0, p1, bias1, rmat, w2ext, w2dst):
    B = 1000
    return pl.pallas_call(
        _tc2_body,
        grid=(_N // B,),
        in_specs=[
            pl.BlockSpec((B, 144), lambda i: (i, 0)),
            pl.BlockSpec((B, 144), lambda i: (i, 0)),
            pl.BlockSpec((1, 128), lambda i: (0, 0)),
            pl.BlockSpec((8, 128), lambda i: (0, 0)),
            pl.BlockSpec((128, 80), lambda i: (0, 0)),
            pl.BlockSpec((128, 16), lambda i: (0, 0)),
        ],
        out_specs=[
            pl.BlockSpec((B, 80), lambda i: (i, 0)),
            pl.BlockSpec((B, 16), lambda i: (i, 0)),
        ],
        out_shape=[
            jax.ShapeDtypeStruct((_N, 80), jnp.float32),
            jax.ShapeDtypeStruct((_N, 16), jnp.float32),
        ],
    )(p0, p1, bias1, rmat, w2ext, w2dst)


def _tc3_body(q0_ref, q1_ref, b2_ref, out_ref):
    acc = q0_ref[...] + q1_ref[...]
    feats = acc[:, :64]
    den = acc[:, 64:65]
    out_ref[...] = feats / (den + 1e-16) + b2_ref[...]


def _tc3(q0, q1, bias2):
    B = 1000
    return pl.pallas_call(
        _tc3_body,
        grid=(_N // B,),
        in_specs=[
            pl.BlockSpec((B, 80), lambda i: (i, 0)),
            pl.BlockSpec((B, 80), lambda i: (i, 0)),
            pl.BlockSpec((1, 64), lambda i: (0, 0)),
        ],
        out_specs=pl.BlockSpec((B, 64), lambda i: (i, 0)),
        out_shape=jax.ShapeDtypeStruct((_N, 64), jnp.float32),
    )(q0, q1, bias2)


def _sc_edge(F, H, FW, n_chunks, e_tot, hx, adst, src, dst):
    CH = _CHUNK
    rows_per_tile = _NACC // _NSUB
    zrows = 25
    seg = F // H
    nseg = F // 16
    mesh = plsc.VectorSubcoreMesh(core_axis_name="c", subcore_axis_name="s")

    def body(hx_hbm, adst_hbm, src_hbm, dst_hbm, out0, out1,
             sidx, didx, didx_s, rows, drows, wrow, acc, gsem, isem, ssem):
        cid = lax.axis_index("c")
        sid = lax.axis_index("s")
        iota = lax.iota(jnp.int32, 16)
        zero16 = jnp.zeros((16,), jnp.float32)

        for r in range(zrows):
            for cb in range(FW // 16):
                wrow[0][r, pl.ds(cb * 16, 16)] = zero16

        def zinit(i, c):
            pltpu.sync_copy(wrow[0].at[pl.ds(0, zrows), :],
                            acc.at[pl.ds(sid * rows_per_tile + i * zrows, zrows), :])
            return c
        lax.fori_loop(0, rows_per_tile // zrows, zinit, 0)

        for b in range(2):
            for g in range(CH // 16):
                ridx = iota + g * 16
                for c in range(F + H, FW):
                    plsc.store_scatter(wrow[b], [ridx, jnp.full((16,), c, jnp.int32)], zero16)

        plsc.subcore_barrier()

        tile_chunk0 = (cid * _NSUB + sid) * n_chunks

        def issue_idx(b, t):
            base = (tile_chunk0 + t) * CH
            pltpu.async_copy(src_hbm.at[pl.ds(base, CH)], sidx[b], isem[b])
            pltpu.async_copy(dst_hbm.at[pl.ds(base, CH)], didx[b], isem[b])

        def drain_idx(b):
            pltpu.make_async_copy(src_hbm.at[pl.ds(0, CH)], sidx[b], isem[b]).wait()
            pltpu.make_async_copy(dst_hbm.at[pl.ds(0, CH)], didx[b], isem[b]).wait()

        def issue_rows(b):
            pltpu.async_copy(hx_hbm.at[sidx[b]], rows[b], gsem[b])
            pltpu.async_copy(adst_hbm.at[didx[b]], drows[b], gsem[b])

        def drain_scatter(b):
            pltpu.make_async_copy(wrow[b], acc.at[didx_s[b]], ssem[b]).wait()

        def process(b, t):
            @pl.when(t >= 2)
            def _():
                drain_scatter(b)
            pltpu.make_async_copy(hx_hbm.at[sidx[b]], rows[b], gsem[b]).wait()
            pltpu.make_async_copy(adst_hbm.at[didx[b]], drows[b], gsem[b]).wait()

            for k in range(CH // 16):
                didx_s[b][pl.ds(k * 16, 16)] = didx[b][pl.ds(k * 16, 16)]

            @pl.when(t + 2 < n_chunks)
            def _():
                issue_idx(b, t + 2)

            base = (tile_chunk0 + t) * CH
            for g in range(CH // 16):
                ridx = iota + g * 16
                valid = (ridx + base) < e_tot
                for h in range(H):
                    col = jnp.full((16,), F + h, jnp.int32)
                    asrc = plsc.load_gather(rows[b], [ridx, col])
                    adv = plsc.load_gather(drows[b], [ridx, jnp.full((16,), h, jnp.int32)])
                    a = asrc + adv
                    a = jnp.where(a > 0.0, a, 0.2 * a)
                    w = jnp.where(valid, jnp.exp(a), 0.0)
                    plsc.store_scatter(wrow[b], [ridx, col], w)

            @plsc.parallel_loop(0, CH, step=1, unroll=8)
            def _feat(e):
                re_ = jnp.full((16,), e, jnp.int32)
                wb0 = plsc.load_gather(wrow[b], [re_, jnp.full((16,), F, jnp.int32)])
                for s in range(nseg):
                    h = (s * 16) // seg
                    if h == 0:
                        wb = wb0
                    else:
                        wb = plsc.load_gather(wrow[b], [re_, jnp.full((16,), F + h, jnp.int32)])
                    cseg = iota + s * 16
                    hv = plsc.load_gather(rows[b], [re_, cseg])
                    plsc.store_scatter(wrow[b], [re_, cseg], hv * wb)

            pltpu.async_copy(wrow[b], acc.at[didx_s[b]], ssem[b], add=True)

            @pl.when(t + 2 < n_chunks)
            def _():
                drain_idx(b)
                issue_rows(b)

        issue_idx(0, 0)
        issue_idx(1, 1)
        drain_idx(0)
        issue_rows(0)
        drain_idx(1)
        issue_rows(1)

        def chunk_pair(i, carry):
            for b in range(2):
                process(b, 2 * i + b)
            return carry

        lax.fori_loop(0, n_chunks // 2, chunk_pair, 0)
        drain_scatter(0)
        drain_scatter(1)
        plsc.subcore_barrier()

        def copy_body(j, c):
            r0 = sid * rows_per_tile + j * zrows
            pltpu.sync_copy(acc.at[pl.ds(r0, zrows), :], rows[0].at[pl.ds(0, zrows), :])

            @pl.when(cid == 0)
            def _():
                pltpu.sync_copy(rows[0].at[pl.ds(0, zrows), :], out0.at[pl.ds(r0, zrows), :])

            @pl.when(cid == 1)
            def _():
                pltpu.sync_copy(rows[0].at[pl.ds(0, zrows), :], out1.at[pl.ds(r0, zrows), :])
            return c
        lax.fori_loop(0, rows_per_tile // zrows, copy_body, 0)

    return pl.kernel(
        body,
        out_type=[
            jax.ShapeDtypeStruct((_NACC, FW), jnp.float32),
            jax.ShapeDtypeStruct((_NACC, FW), jnp.float32),
        ],
        mesh=mesh,
        compiler_params=pltpu.CompilerParams(
            use_tc_tiling_on_sc=False, needs_layout_passes=False),
        scratch_types=[
            [pltpu.VMEM((CH,), jnp.int32)] * 2,
            [pltpu.VMEM((CH,), jnp.int32)] * 2,
            [pltpu.VMEM((CH,), jnp.int32)] * 2,
            [pltpu.VMEM((CH, FW), jnp.float32)] * 2,
            [pltpu.VMEM((CH, 16), jnp.float32)] * 2,
            [pltpu.VMEM((CH, FW), jnp.float32)] * 2,
            pltpu.VMEM_SHARED((_NACC, FW), jnp.float32),
            [pltpu.SemaphoreType.DMA] * 2,
            [pltpu.SemaphoreType.DMA] * 2,
            [pltpu.SemaphoreType.DMA] * 2,
        ],
    )(hx, adst, src, dst)


def kernel(x, edge_index, W1, att_src1, att_dst1, bias1, W2, att_src2, att_dst2, bias2):
    n = x.shape[0]
    loop = jnp.arange(n, dtype=edge_index.dtype)
    ei = jnp.concatenate([edge_index, jnp.stack([loop, loop])], axis=1)
    e_tot = ei.shape[1]
    per = _CHUNK * _NCORE * _NSUB
    n_chunks = -(-e_tot // per)
    e_pad = n_chunks * per
    src = jnp.concatenate([ei[0], jnp.zeros((e_pad - e_tot,), jnp.int32)])
    dst = jnp.concatenate([ei[1], jnp.zeros((e_pad - e_tot,), jnp.int32)])

    heads, hid = att_src1.shape
    eye = jnp.eye(heads, dtype=jnp.float32)
    asrc_blk = (eye[:, None, :] * att_src1[:, :, None]).reshape(heads * hid, heads)
    adst_blk = (eye[:, None, :] * att_dst1[:, :, None]).reshape(heads * hid, heads)
    wext1 = jnp.concatenate(
        [W1, W1 @ asrc_blk, jnp.zeros((W1.shape[0], 8), jnp.float32)], axis=1)
    wdst1 = jnp.concatenate(
        [W1 @ adst_blk, jnp.zeros((W1.shape[0], 8), jnp.float32)], axis=1)
    rmat = (eye[:, None, :] * jnp.ones((heads, hid, 1), jnp.float32)).reshape(
        heads * hid, heads).T
    w2ext = jnp.concatenate(
        [W2, W2 @ att_src2.T, jnp.zeros((W2.shape[0], 15), jnp.float32)], axis=1)
    w2dst = jnp.concatenate(
        [W2 @ att_dst2.T, jnp.zeros((W2.shape[0], 15), jnp.float32)], axis=1)

    hx, ad1 = _tc1(x, wext1, wdst1)
    p0, p1 = _sc_edge(128, 8, 144, n_chunks, e_tot, hx, ad1, src, dst)
    hx2, ad2 = _tc2(p0, p1, bias1.reshape(1, 128), rmat, w2ext, w2dst)
    q0, q1 = _sc_edge(64, 1, 80, n_chunks, e_tot, hx2, ad2, src, dst)
    return _tc3(q0, q1, bias2.reshape(1, 64))


# speedup vs baseline: 1.0138x; 1.0138x over previous
import jax
import jax.numpy as jnp
from jax import lax
from jax.experimental import pallas as pl
from jax.experimental.pallas import tpu as pltpu
from jax.experimental.pallas import tpu_sc as plsc

_N = 10000
_NCORE = 2
_NSUB = 16
_CHUNK = 64
_NACC = 10000


def _tc1_body(x_ref, we_ref, wd_ref, hx_ref, ad_ref):
    xb = x_ref[...]
    hx_ref[...] = jnp.dot(xb, we_ref[...], preferred_element_type=jnp.float32)
    ad_ref[...] = jnp.dot(xb, wd_ref[...], preferred_element_type=jnp.float32)


def _tc1(x, wext, wdst):
    B = 1000
    return pl.pallas_call(
        _tc1_body,
        grid=(_N // B,),
        in_specs=[
            pl.BlockSpec((B, 128), lambda i: (i, 0)),
            pl.BlockSpec((128, 144), lambda i: (0, 0)),
            pl.BlockSpec((128, 16), lambda i: (0, 0)),
        ],
        out_specs=[
            pl.BlockSpec((B, 144), lambda i: (i, 0)),
            pl.BlockSpec((B, 16), lambda i: (i, 0)),
        ],
        out_shape=[
            jax.ShapeDtypeStruct((_N, 144), jnp.float32),
            jax.ShapeDtypeStruct((_N, 16), jnp.float32),
        ],
    )(x, wext, wdst)


def _tc2_body(p0_ref, p1_ref, b1_ref, rmat_ref, w2e_ref, w2d_ref, hx2_ref, ad2_ref):
    acc = p0_ref[...] + p1_ref[...]
    feats = acc[:, :128]
    den = acc[:, 128:136]
    denb = jnp.dot(den, rmat_ref[...], preferred_element_type=jnp.float32) + 1e-16
    z = feats / denb + b1_ref[...]
    z = jnp.where(z > 0.0, z, jnp.exp(jnp.minimum(z, 0.0)) - 1.0)
    hx2_ref[...] = jnp.dot(z, w2e_ref[...], preferred_element_type=jnp.float32)
    ad2_ref[...] = jnp.dot(z, w2d_ref[...], preferred_element_type=jnp.float32)


def _tc2(p0, p1, bias1, rmat, w2ext, w2dst):
    B = 1000
    return pl.pallas_call(
        _tc2_body,
        grid=(_N // B,),
        in_specs=[
            pl.BlockSpec((B, 144), lambda i: (i, 0)),
            pl.BlockSpec((B, 144), lambda i: (i, 0)),
            pl.BlockSpec((1, 128), lambda i: (0, 0)),
            pl.BlockSpec((8, 128), lambda i: (0, 0)),
            pl.BlockSpec((128, 80), lambda i: (0, 0)),
            pl.BlockSpec((128, 16), lambda i: (0, 0)),
        ],
        out_specs=[
            pl.BlockSpec((B, 80), lambda i: (i, 0)),
            pl.BlockSpec((B, 16), lambda i: (i, 0)),
        ],
        out_shape=[
            jax.ShapeDtypeStruct((_N, 80), jnp.float32),
            jax.ShapeDtypeStruct((_N, 16), jnp.float32),
        ],
    )(p0, p1, bias1, rmat, w2ext, w2dst)


def _tc3_body(q0_ref, q1_ref, b2_ref, out_ref):
    acc = q0_ref[...] + q1_ref[...]
    feats = acc[:, :64]
    den = acc[:, 64:65]
    out_ref[...] = feats / (den + 1e-16) + b2_ref[...]


def _tc3(q0, q1, bias2):
    B = 1000
    return pl.pallas_call(
        _tc3_body,
        grid=(_N // B,),
        in_specs=[
            pl.BlockSpec((B, 80), lambda i: (i, 0)),
            pl.BlockSpec((B, 80), lambda i: (i, 0)),
            pl.BlockSpec((1, 64), lambda i: (0, 0)),
        ],
        out_specs=pl.BlockSpec((B, 64), lambda i: (i, 0)),
        out_shape=jax.ShapeDtypeStruct((_N, 64), jnp.float32),
    )(q0, q1, bias2)


def _sc_edge(F, H, FW, n_chunks, e_tot, hx, adst, src, dst):
    CH = _CHUNK
    rows_per_tile = _NACC // _NSUB
    zrows = 25
    seg = F // H
    nseg = F // 16
    mesh = plsc.VectorSubcoreMesh(core_axis_name="c", subcore_axis_name="s")

    def body(hx_hbm, adst_hbm, src_hbm, dst_hbm, out0, out1,
             sidx, didx, didx_s, rows, drows, wrow, acc, gsem, isem, ssem):
        cid = lax.axis_index("c")
        sid = lax.axis_index("s")
        iota = lax.iota(jnp.int32, 16)
        zero16 = jnp.zeros((16,), jnp.float32)

        for r in range(zrows):
            for cb in range(FW // 16):
                wrow[0][r, pl.ds(cb * 16, 16)] = zero16

        def zinit(i, c):
            pltpu.sync_copy(wrow[0].at[pl.ds(0, zrows), :],
                            acc.at[pl.ds(sid * rows_per_tile + i * zrows, zrows), :])
            return c
        lax.fori_loop(0, rows_per_tile // zrows, zinit, 0)

        for b in range(2):
            for g in range(CH // 16):
                ridx = iota + g * 16
                for c in range(F + H, FW):
                    plsc.store_scatter(wrow[b], [ridx, jnp.full((16,), c, jnp.int32)], zero16)

        plsc.subcore_barrier()

        tile_chunk0 = (cid * _NSUB + sid) * n_chunks

        def issue_idx(b, t):
            base = (tile_chunk0 + t) * CH
            pltpu.async_copy(src_hbm.at[pl.ds(base, CH)], sidx[b], isem[b])
            pltpu.async_copy(dst_hbm.at[pl.ds(base, CH)], didx[b], isem[b])

        def drain_idx(b):
            pltpu.make_async_copy(src_hbm.at[pl.ds(0, CH)], sidx[b], isem[b]).wait()
            pltpu.make_async_copy(dst_hbm.at[pl.ds(0, CH)], didx[b], isem[b]).wait()

        def issue_rows(b):
            pltpu.async_copy(hx_hbm.at[sidx[b]], rows[b], gsem[b])
            pltpu.async_copy(adst_hbm.at[didx[b]], drows[b], gsem[b])

        def drain_scatter(b):
            pltpu.make_async_copy(wrow[b], acc.at[didx_s[b]], ssem[b]).wait()

        def process(b, t):
            @pl.when(t >= 2)
            def _():
                drain_scatter(b)
            pltpu.make_async_copy(hx_hbm.at[sidx[b]], rows[b], gsem[b]).wait()
            pltpu.make_async_copy(adst_hbm.at[didx[b]], drows[b], gsem[b]).wait()

            for k in range(CH // 16):
                didx_s[b][pl.ds(k * 16, 16)] = didx[b][pl.ds(k * 16, 16)]

            @pl.when(t + 2 < n_chunks)
            def _():
                issue_idx(b, t + 2)

            base = (tile_chunk0 + t) * CH
            for g in range(CH // 16):
                ridx = iota + g * 16
                valid = (ridx + base) < e_tot
                for h in range(H):
                    col = jnp.full((16,), F + h, jnp.int32)
                    asrc = plsc.load_gather(rows[b], [ridx, col])
                    adv = plsc.load_gather(drows[b], [ridx, jnp.full((16,), h, jnp.int32)])
                    a = asrc + adv
                    a = jnp.where(a > 0.0, a, 0.2 * a)
                    w = jnp.where(valid, jnp.exp(a), 0.0)
                    plsc.store_scatter(wrow[b], [ridx, col], w)

            @plsc.parallel_loop(0, CH, step=1, unroll=4)
            def _feat(e):
                re_ = jnp.full((16,), e, jnp.int32)
                wb0 = plsc.load_gather(wrow[b], [re_, jnp.full((16,), F, jnp.int32)])
                for s in range(nseg):
                    h = (s * 16) // seg
                    if h == 0:
                        wb = wb0
                    else:
                        wb = plsc.load_gather(wrow[b], [re_, jnp.full((16,), F + h, jnp.int32)])
                    cseg = iota + s * 16
                    hv = plsc.load_gather(rows[b], [re_, cseg])
                    plsc.store_scatter(wrow[b], [re_, cseg], hv * wb)

            pltpu.async_copy(wrow[b], acc.at[didx_s[b]], ssem[b], add=True)

            @pl.when(t + 2 < n_chunks)
            def _():
                drain_idx(b)
                issue_rows(b)

        issue_idx(0, 0)
        issue_idx(1, 1)
        drain_idx(0)
        issue_rows(0)
        drain_idx(1)
        issue_rows(1)

        def chunk_pair(i, carry):
            for b in range(2):
                process(b, 2 * i + b)
            return carry

        lax.fori_loop(0, n_chunks // 2, chunk_pair, 0)
        drain_scatter(0)
        drain_scatter(1)
        plsc.subcore_barrier()

        def copy_body(j, c):
            r0 = sid * rows_per_tile + j * zrows
            pltpu.sync_copy(acc.at[pl.ds(r0, zrows), :], rows[0].at[pl.ds(0, zrows), :])

            @pl.when(cid == 0)
            def _():
                pltpu.sync_copy(rows[0].at[pl.ds(0, zrows), :], out0.at[pl.ds(r0, zrows), :])

            @pl.when(cid == 1)
            def _():
                pltpu.sync_copy(rows[0].at[pl.ds(0, zrows), :], out1.at[pl.ds(r0, zrows), :])
            return c
        lax.fori_loop(0, rows_per_tile // zrows, copy_body, 0)

    return pl.kernel(
        body,
        out_type=[
            jax.ShapeDtypeStruct((_NACC, FW), jnp.float32),
            jax.ShapeDtypeStruct((_NACC, FW), jnp.float32),
        ],
        mesh=mesh,
        compiler_params=pltpu.CompilerParams(
            use_tc_tiling_on_sc=False, needs_layout_passes=False),
        scratch_types=[
            [pltpu.VMEM((CH,), jnp.int32)] * 2,
            [pltpu.VMEM((CH,), jnp.int32)] * 2,
            [pltpu.VMEM((CH,), jnp.int32)] * 2,
            [pltpu.VMEM((CH, FW), jnp.float32)] * 2,
            [pltpu.VMEM((CH, 16), jnp.float32)] * 2,
            [pltpu.VMEM((CH, FW), jnp.float32)] * 2,
            pltpu.VMEM_SHARED((_NACC, FW), jnp.float32),
            [pltpu.SemaphoreType.DMA] * 2,
            [pltpu.SemaphoreType.DMA] * 2,
            [pltpu.SemaphoreType.DMA] * 2,
        ],
    )(hx, adst, src, dst)


def kernel(x, edge_index, W1, att_src1, att_dst1, bias1, W2, att_src2, att_dst2, bias2):
    n = x.shape[0]
    loop = jnp.arange(n, dtype=edge_index.dtype)
    ei = jnp.concatenate([edge_index, jnp.stack([loop, loop])], axis=1)
    e_tot = ei.shape[1]
    per = _CHUNK * _NCORE * _NSUB
    n_chunks = -(-e_tot // per)
    e_pad = n_chunks * per
    src = jnp.concatenate([ei[0], jnp.zeros((e_pad - e_tot,), jnp.int32)])
    dst = jnp.concatenate([ei[1], jnp.zeros((e_pad - e_tot,), jnp.int32)])

    heads, hid = att_src1.shape
    eye = jnp.eye(heads, dtype=jnp.float32)
    asrc_blk = (eye[:, None, :] * att_src1[:, :, None]).reshape(heads * hid, heads)
    adst_blk = (eye[:, None, :] * att_dst1[:, :, None]).reshape(heads * hid, heads)
    wext1 = jnp.concatenate(
        [W1, W1 @ asrc_blk, jnp.zeros((W1.shape[0], 8), jnp.float32)], axis=1)
    wdst1 = jnp.concatenate(
        [W1 @ adst_blk, jnp.zeros((W1.shape[0], 8), jnp.float32)], axis=1)
    rmat = (eye[:, None, :] * jnp.ones((heads, hid, 1), jnp.float32)).reshape(
        heads * hid, heads).T
    w2ext = jnp.concatenate(
        [W2, W2 @ att_src2.T, jnp.zeros((W2.shape[0], 15), jnp.float32)], axis=1)
    w2dst = jnp.concatenate(
        [W2 @ att_dst2.T, jnp.zeros((W2.shape[0], 15), jnp.float32)], axis=1)

    hx, ad1 = _tc1(x, wext1, wdst1)
    p0, p1 = _sc_edge(128, 8, 144, n_chunks, e_tot, hx, ad1, src, dst)
    hx2, ad2 = _tc2(p0, p1, bias1.reshape(1, 128), rmat, w2ext, w2dst)
    q0, q1 = _sc_edge(64, 1, 80, n_chunks, e_tot, hx2, ad2, src, dst)
    return _tc3(q0, q1, bias2.reshape(1, 64))


# speedup vs baseline: 1.1124x; 1.0973x over previous
import jax
import jax.numpy as jnp
from jax import lax
from jax.experimental import pallas as pl
from jax.experimental.pallas import tpu as pltpu
from jax.experimental.pallas import tpu_sc as plsc

_N = 10000
_NCORE = 2
_NSUB = 16
_CHUNK = 64
_NACC = 10000
_GDN = lax.GatherDimensionNumbers(
    offset_dims=(), collapsed_slice_dims=(0,), start_index_map=(0,))


def _tc1_body(x_ref, we_ref, wd_ref, hx_ref, ad_ref):
    xb = x_ref[...]
    hx_ref[...] = jnp.dot(xb, we_ref[...], preferred_element_type=jnp.float32)
    ad_ref[...] = jnp.dot(xb, wd_ref[...], preferred_element_type=jnp.float32)


def _tc1(x, wext, wdst):
    B = 1000
    return pl.pallas_call(
        _tc1_body,
        grid=(_N // B,),
        in_specs=[
            pl.BlockSpec((B, 128), lambda i: (i, 0)),
            pl.BlockSpec((128, 144), lambda i: (0, 0)),
            pl.BlockSpec((128, 16), lambda i: (0, 0)),
        ],
        out_specs=[
            pl.BlockSpec((B, 144), lambda i: (i, 0)),
            pl.BlockSpec((B, 16), lambda i: (i, 0)),
        ],
        out_shape=[
            jax.ShapeDtypeStruct((_N, 144), jnp.float32),
            jax.ShapeDtypeStruct((_N, 16), jnp.float32),
        ],
    )(x, wext, wdst)


def _tc2_body(p0_ref, p1_ref, b1_ref, rmat_ref, w2e_ref, w2d_ref, hx2_ref, ad2_ref):
    acc = p0_ref[...] + p1_ref[...]
    feats = acc[:, :128]
    den = acc[:, 128:136]
    denb = jnp.dot(den, rmat_ref[...], preferred_element_type=jnp.float32) + 1e-16
    z = feats / denb + b1_ref[...]
    z = jnp.where(z > 0.0, z, jnp.exp(jnp.minimum(z, 0.0)) - 1.0)
    hx2_ref[...] = jnp.dot(z, w2e_ref[...], preferred_element_type=jnp.float32)
    ad2_ref[...] = jnp.dot(z, w2d_ref[...], preferred_element_type=jnp.float32)


def _tc2(p0, p1, bias1, rmat, w2ext, w2dst):
    B = 1000
    return pl.pallas_call(
        _tc2_body,
        grid=(_N // B,),
        in_specs=[
            pl.BlockSpec((B, 144), lambda i: (i, 0)),
            pl.BlockSpec((B, 144), lambda i: (i, 0)),
            pl.BlockSpec((1, 128), lambda i: (0, 0)),
            pl.BlockSpec((8, 128), lambda i: (0, 0)),
            pl.BlockSpec((128, 80), lambda i: (0, 0)),
            pl.BlockSpec((128, 16), lambda i: (0, 0)),
        ],
        out_specs=[
            pl.BlockSpec((B, 80), lambda i: (i, 0)),
            pl.BlockSpec((B, 16), lambda i: (i, 0)),
        ],
        out_shape=[
            jax.ShapeDtypeStruct((_N, 80), jnp.float32),
            jax.ShapeDtypeStruct((_N, 16), jnp.float32),
        ],
    )(p0, p1, bias1, rmat, w2ext, w2dst)


def _tc3_body(q0_ref, q1_ref, b2_ref, out_ref):
    acc = q0_ref[...] + q1_ref[...]
    feats = acc[:, :64]
    den = acc[:, 64:65]
    out_ref[...] = feats / (den + 1e-16) + b2_ref[...]


def _tc3(q0, q1, bias2):
    B = 1000
    return pl.pallas_call(
        _tc3_body,
        grid=(_N // B,),
        in_specs=[
            pl.BlockSpec((B, 80), lambda i: (i, 0)),
            pl.BlockSpec((B, 80), lambda i: (i, 0)),
            pl.BlockSpec((1, 64), lambda i: (0, 0)),
        ],
        out_specs=pl.BlockSpec((B, 64), lambda i: (i, 0)),
        out_shape=jax.ShapeDtypeStruct((_N, 64), jnp.float32),
    )(q0, q1, bias2)


def _sc_edge(F, H, FW, n_chunks, e_tot, hx, adst, src, dst):
    CH = _CHUNK
    rows_per_tile = _NACC // _NSUB
    zrows = 25
    seg = F // H
    nseg = F // 16
    mesh = plsc.VectorSubcoreMesh(core_axis_name="c", subcore_axis_name="s")

    def body(hx_hbm, adst_hbm, src_hbm, dst_hbm, out0, out1,
             sidx, didx, didx_s, rows, drows, wrow, acc, gsem, isem, ssem):
        cid = lax.axis_index("c")
        sid = lax.axis_index("s")
        iota = lax.iota(jnp.int32, 16)
        zero16 = jnp.zeros((16,), jnp.float32)

        for r in range(zrows):
            for cb in range(FW // 16):
                wrow[0][r, pl.ds(cb * 16, 16)] = zero16

        def zinit(i, c):
            pltpu.sync_copy(wrow[0].at[pl.ds(0, zrows), :],
                            acc.at[pl.ds(sid * rows_per_tile + i * zrows, zrows), :])
            return c
        lax.fori_loop(0, rows_per_tile // zrows, zinit, 0)

        plsc.subcore_barrier()

        tile_chunk0 = (cid * _NSUB + sid) * n_chunks

        def issue_idx(b, t):
            base = (tile_chunk0 + t) * CH
            pltpu.async_copy(src_hbm.at[pl.ds(base, CH)], sidx[b], isem[b])
            pltpu.async_copy(dst_hbm.at[pl.ds(base, CH)], didx[b], isem[b])

        def drain_idx(b):
            pltpu.make_async_copy(src_hbm.at[pl.ds(0, CH)], sidx[b], isem[b]).wait()
            pltpu.make_async_copy(dst_hbm.at[pl.ds(0, CH)], didx[b], isem[b]).wait()

        def issue_rows(b):
            pltpu.async_copy(hx_hbm.at[sidx[b]], rows[b], gsem[b])
            pltpu.async_copy(adst_hbm.at[didx[b]], drows[b], gsem[b])

        def drain_scatter(b):
            pltpu.make_async_copy(wrow[b], acc.at[didx_s[b]], ssem[b]).wait()

        def process(b, t):
            @pl.when(t >= 2)
            def _():
                drain_scatter(b)
            pltpu.make_async_copy(hx_hbm.at[sidx[b]], rows[b], gsem[b]).wait()
            pltpu.make_async_copy(adst_hbm.at[didx[b]], drows[b], gsem[b]).wait()

            for k in range(CH // 16):
                didx_s[b][pl.ds(k * 16, 16)] = didx[b][pl.ds(k * 16, 16)]

            @pl.when(t + 2 < n_chunks)
            def _():
                issue_idx(b, t + 2)

            base = (tile_chunk0 + t) * CH

            @plsc.parallel_loop(0, CH, step=1, unroll=4)
            def _edge(e):
                re_ = jnp.full((16,), e, jnp.int32)
                av = plsc.load_gather(rows[b], [re_, iota + F])
                dv = plsc.load_gather(drows[b], [re_, iota])
                a = av + dv
                a = jnp.where(a > 0.0, a, 0.2 * a)
                w = jnp.where(base + e < e_tot, jnp.exp(a), jnp.zeros((16,), jnp.float32))
                plsc.store_scatter(wrow[b], [re_, iota + F], w)
                for s in range(nseg):
                    h = (s * 16) // seg
                    wb = lax.gather(
                        w, jnp.full((16, 1), h, jnp.int32), _GDN, slice_sizes=(1,),
                        mode=lax.GatherScatterMode.PROMISE_IN_BOUNDS)
                    cseg = iota + s * 16
                    hv = plsc.load_gather(rows[b], [re_, cseg])
                    plsc.store_scatter(wrow[b], [re_, cseg], hv * wb)

            pltpu.async_copy(wrow[b], acc.at[didx_s[b]], ssem[b], add=True)

            @pl.when(t + 2 < n_chunks)
            def _():
                drain_idx(b)
                issue_rows(b)

        issue_idx(0, 0)
        issue_idx(1, 1)
        drain_idx(0)
        issue_rows(0)
        drain_idx(1)
        issue_rows(1)

        def chunk_pair(i, carry):
            for b in range(2):
                process(b, 2 * i + b)
            return carry

        lax.fori_loop(0, n_chunks // 2, chunk_pair, 0)
        drain_scatter(0)
        drain_scatter(1)
        plsc.subcore_barrier()

        def copy_body(j, c):
            r0 = sid * rows_per_tile + j * zrows
            pltpu.sync_copy(acc.at[pl.ds(r0, zrows), :], rows[0].at[pl.ds(0, zrows), :])

            @pl.when(cid == 0)
            def _():
                pltpu.sync_copy(rows[0].at[pl.ds(0, zrows), :], out0.at[pl.ds(r0, zrows), :])

            @pl.when(cid == 1)
            def _():
                pltpu.sync_copy(rows[0].at[pl.ds(0, zrows), :], out1.at[pl.ds(r0, zrows), :])
            return c
        lax.fori_loop(0, rows_per_tile // zrows, copy_body, 0)

    return pl.kernel(
        body,
        out_type=[
            jax.ShapeDtypeStruct((_NACC, FW), jnp.float32),
            jax.ShapeDtypeStruct((_NACC, FW), jnp.float32),
        ],
        mesh=mesh,
        compiler_params=pltpu.CompilerParams(
            use_tc_tiling_on_sc=False, needs_layout_passes=False),
        scratch_types=[
            [pltpu.VMEM((CH,), jnp.int32)] * 2,
            [pltpu.VMEM((CH,), jnp.int32)] * 2,
            [pltpu.VMEM((CH,), jnp.int32)] * 2,
            [pltpu.VMEM((CH, FW), jnp.float32)] * 2,
            [pltpu.VMEM((CH, 16), jnp.float32)] * 2,
            [pltpu.VMEM((CH, FW), jnp.float32)] * 2,
            pltpu.VMEM_SHARED((_NACC, FW), jnp.float32),
            [pltpu.SemaphoreType.DMA] * 2,
            [pltpu.SemaphoreType.DMA] * 2,
            [pltpu.SemaphoreType.DMA] * 2,
        ],
    )(hx, adst, src, dst)


def kernel(x, edge_index, W1, att_src1, att_dst1, bias1, W2, att_src2, att_dst2, bias2):
    n = x.shape[0]
    loop = jnp.arange(n, dtype=edge_index.dtype)
    ei = jnp.concatenate([edge_index, jnp.stack([loop, loop])], axis=1)
    e_tot = ei.shape[1]
    per = _CHUNK * _NCORE * _NSUB
    n_chunks = -(-e_tot // per)
    e_pad = n_chunks * per
    src = jnp.concatenate([ei[0], jnp.zeros((e_pad - e_tot,), jnp.int32)])
    dst = jnp.concatenate([ei[1], jnp.zeros((e_pad - e_tot,), jnp.int32)])

    heads, hid = att_src1.shape
    eye = jnp.eye(heads, dtype=jnp.float32)
    asrc_blk = (eye[:, None, :] * att_src1[:, :, None]).reshape(heads * hid, heads)
    adst_blk = (eye[:, None, :] * att_dst1[:, :, None]).reshape(heads * hid, heads)
    wext1 = jnp.concatenate(
        [W1, W1 @ asrc_blk, jnp.zeros((W1.shape[0], 8), jnp.float32)], axis=1)
    wdst1 = jnp.concatenate(
        [W1 @ adst_blk, jnp.zeros((W1.shape[0], 8), jnp.float32)], axis=1)
    rmat = (eye[:, None, :] * jnp.ones((heads, hid, 1), jnp.float32)).reshape(
        heads * hid, heads).T
    w2ext = jnp.concatenate(
        [W2, W2 @ att_src2.T, jnp.zeros((W2.shape[0], 15), jnp.float32)], axis=1)
    w2dst = jnp.concatenate(
        [W2 @ att_dst2.T, jnp.zeros((W2.shape[0], 15), jnp.float32)], axis=1)

    hx, ad1 = _tc1(x, wext1, wdst1)
    p0, p1 = _sc_edge(128, 8, 144, n_chunks, e_tot, hx, ad1, src, dst)
    hx2, ad2 = _tc2(p0, p1, bias1.reshape(1, 128), rmat, w2ext, w2dst)
    q0, q1 = _sc_edge(64, 1, 80, n_chunks, e_tot, hx2, ad2, src, dst)
    return _tc3(q0, q1, bias2.reshape(1, 64))


# speedup vs baseline: 1.1161x; 1.0034x over previous
import jax
import jax.numpy as jnp
from jax import lax
from jax.experimental import pallas as pl
from jax.experimental.pallas import tpu as pltpu
from jax.experimental.pallas import tpu_sc as plsc

_N = 10000
_NCORE = 2
_NSUB = 16
_CHUNK = 64
_NACC = 10000
_GDN = lax.GatherDimensionNumbers(
    offset_dims=(), collapsed_slice_dims=(0,), start_index_map=(0,))


def _tc1_body(x_ref, we_ref, wd_ref, hx_ref, ad_ref):
    xb = x_ref[...]
    hx_ref[...] = jnp.dot(xb, we_ref[...], preferred_element_type=jnp.float32)
    ad_ref[...] = jnp.dot(xb, wd_ref[...], preferred_element_type=jnp.float32)


def _tc1(x, wext, wdst):
    B = 1000
    return pl.pallas_call(
        _tc1_body,
        grid=(_N // B,),
        in_specs=[
            pl.BlockSpec((B, 128), lambda i: (i, 0)),
            pl.BlockSpec((128, 144), lambda i: (0, 0)),
            pl.BlockSpec((128, 16), lambda i: (0, 0)),
        ],
        out_specs=[
            pl.BlockSpec((B, 144), lambda i: (i, 0)),
            pl.BlockSpec((B, 16), lambda i: (i, 0)),
        ],
        out_shape=[
            jax.ShapeDtypeStruct((_N, 144), jnp.float32),
            jax.ShapeDtypeStruct((_N, 16), jnp.float32),
        ],
    )(x, wext, wdst)


def _tc2_body(p0_ref, p1_ref, b1_ref, rmat_ref, w2e_ref, w2d_ref, hx2_ref, ad2_ref):
    acc = p0_ref[...] + p1_ref[...]
    feats = acc[:, :128]
    den = acc[:, 128:136]
    denb = jnp.dot(den, rmat_ref[...], preferred_element_type=jnp.float32) + 1e-16
    z = feats / denb + b1_ref[...]
    z = jnp.where(z > 0.0, z, jnp.exp(jnp.minimum(z, 0.0)) - 1.0)
    hx2_ref[...] = jnp.dot(z, w2e_ref[...], preferred_element_type=jnp.float32)
    ad2_ref[...] = jnp.dot(z, w2d_ref[...], preferred_element_type=jnp.float32)


def _tc2(p0, p1, bias1, rmat, w2ext, w2dst):
    B = 1000
    return pl.pallas_call(
        _tc2_body,
        grid=(_N // B,),
        in_specs=[
            pl.BlockSpec((B, 144), lambda i: (i, 0)),
            pl.BlockSpec((B, 144), lambda i: (i, 0)),
            pl.BlockSpec((1, 128), lambda i: (0, 0)),
            pl.BlockSpec((8, 128), lambda i: (0, 0)),
            pl.BlockSpec((128, 80), lambda i: (0, 0)),
            pl.BlockSpec((128, 16), lambda i: (0, 0)),
        ],
        out_specs=[
            pl.BlockSpec((B, 80), lambda i: (i, 0)),
            pl.BlockSpec((B, 16), lambda i: (i, 0)),
        ],
        out_shape=[
            jax.ShapeDtypeStruct((_N, 80), jnp.float32),
            jax.ShapeDtypeStruct((_N, 16), jnp.float32),
        ],
    )(p0, p1, bias1, rmat, w2ext, w2dst)


def _tc3_body(q0_ref, q1_ref, b2_ref, out_ref):
    acc = q0_ref[...] + q1_ref[...]
    feats = acc[:, :64]
    den = acc[:, 64:65]
    out_ref[...] = feats / (den + 1e-16) + b2_ref[...]


def _tc3(q0, q1, bias2):
    B = 1000
    return pl.pallas_call(
        _tc3_body,
        grid=(_N // B,),
        in_specs=[
            pl.BlockSpec((B, 80), lambda i: (i, 0)),
            pl.BlockSpec((B, 80), lambda i: (i, 0)),
            pl.BlockSpec((1, 64), lambda i: (0, 0)),
        ],
        out_specs=pl.BlockSpec((B, 64), lambda i: (i, 0)),
        out_shape=jax.ShapeDtypeStruct((_N, 64), jnp.float32),
    )(q0, q1, bias2)


def _sc_edge(F, H, FW, n_chunks, e_tot, hx, adst, src, dst):
    CH = _CHUNK
    rows_per_tile = _NACC // _NSUB
    zrows = 25
    seg = F // H
    nseg = F // 16
    mesh = plsc.VectorSubcoreMesh(core_axis_name="c", subcore_axis_name="s")

    def body(hx_hbm, adst_hbm, src_hbm, dst_hbm, out0, out1,
             sidx, didx, didx_s, rows, drows, wrow, acc, gsem, isem, ssem):
        cid = lax.axis_index("c")
        sid = lax.axis_index("s")
        iota = lax.iota(jnp.int32, 16)
        zero16 = jnp.zeros((16,), jnp.float32)

        for r in range(zrows):
            for cb in range(FW // 16):
                wrow[0][r, pl.ds(cb * 16, 16)] = zero16

        def zinit(i, c):
            pltpu.sync_copy(wrow[0].at[pl.ds(0, zrows), :],
                            acc.at[pl.ds(sid * rows_per_tile + i * zrows, zrows), :])
            return c
        lax.fori_loop(0, rows_per_tile // zrows, zinit, 0)

        plsc.subcore_barrier()

        tile_chunk0 = (cid * _NSUB + sid) * n_chunks

        def issue_idx(b, t):
            base = (tile_chunk0 + t) * CH
            pltpu.async_copy(src_hbm.at[pl.ds(base, CH)], sidx[b], isem[b])
            pltpu.async_copy(dst_hbm.at[pl.ds(base, CH)], didx[b], isem[b])

        def drain_idx(b):
            pltpu.make_async_copy(src_hbm.at[pl.ds(0, CH)], sidx[b], isem[b]).wait()
            pltpu.make_async_copy(dst_hbm.at[pl.ds(0, CH)], didx[b], isem[b]).wait()

        def issue_rows(b):
            pltpu.async_copy(hx_hbm.at[sidx[b]], rows[b], gsem[b])
            pltpu.async_copy(adst_hbm.at[didx[b]], drows[b], gsem[b])

        def drain_scatter(b):
            pltpu.make_async_copy(wrow[b], acc.at[didx_s[b]], ssem[b]).wait()

        def process(b, t):
            @pl.when(t >= 2)
            def _():
                drain_scatter(b)
            pltpu.make_async_copy(hx_hbm.at[sidx[b]], rows[b], gsem[b]).wait()
            pltpu.make_async_copy(adst_hbm.at[didx[b]], drows[b], gsem[b]).wait()

            for k in range(CH // 16):
                didx_s[b][pl.ds(k * 16, 16)] = didx[b][pl.ds(k * 16, 16)]

            @pl.when(t + 2 < n_chunks)
            def _():
                issue_idx(b, t + 2)

            base = (tile_chunk0 + t) * CH

            @plsc.parallel_loop(0, CH, step=1, unroll=8)
            def _edge(e):
                re_ = jnp.full((16,), e, jnp.int32)
                av = plsc.load_gather(rows[b], [re_, iota + F])
                dv = plsc.load_gather(drows[b], [re_, iota])
                a = av + dv
                a = jnp.where(a > 0.0, a, 0.2 * a)
                w = jnp.where(base + e < e_tot, jnp.exp(a), jnp.zeros((16,), jnp.float32))
                plsc.store_scatter(wrow[b], [re_, iota + F], w)
                for s in range(nseg):
                    h = (s * 16) // seg
                    wb = lax.gather(
                        w, jnp.full((16, 1), h, jnp.int32), _GDN, slice_sizes=(1,),
                        mode=lax.GatherScatterMode.PROMISE_IN_BOUNDS)
                    cseg = iota + s * 16
                    hv = plsc.load_gather(rows[b], [re_, cseg])
                    plsc.store_scatter(wrow[b], [re_, cseg], hv * wb)

            pltpu.async_copy(wrow[b], acc.at[didx_s[b]], ssem[b], add=True)

            @pl.when(t + 2 < n_chunks)
            def _():
                drain_idx(b)
                issue_rows(b)

        issue_idx(0, 0)
        issue_idx(1, 1)
        drain_idx(0)
        issue_rows(0)
        drain_idx(1)
        issue_rows(1)

        def chunk_pair(i, carry):
            for b in range(2):
                process(b, 2 * i + b)
            return carry

        lax.fori_loop(0, n_chunks // 2, chunk_pair, 0)
        drain_scatter(0)
        drain_scatter(1)
        plsc.subcore_barrier()

        def copy_body(j, c):
            r0 = sid * rows_per_tile + j * zrows
            pltpu.sync_copy(acc.at[pl.ds(r0, zrows), :], rows[0].at[pl.ds(0, zrows), :])

            @pl.when(cid == 0)
            def _():
                pltpu.sync_copy(rows[0].at[pl.ds(0, zrows), :], out0.at[pl.ds(r0, zrows), :])

            @pl.when(cid == 1)
            def _():
                pltpu.sync_copy(rows[0].at[pl.ds(0, zrows), :], out1.at[pl.ds(r0, zrows), :])
            return c
        lax.fori_loop(0, rows_per_tile // zrows, copy_body, 0)

    return pl.kernel(
        body,
        out_type=[
            jax.ShapeDtypeStruct((_NACC, FW), jnp.float32),
            jax.ShapeDtypeStruct((_NACC, FW), jnp.float32),
        ],
        mesh=mesh,
        compiler_params=pltpu.CompilerParams(
            use_tc_tiling_on_sc=False, needs_layout_passes=False),
        scratch_types=[
            [pltpu.VMEM((CH,), jnp.int32)] * 2,
            [pltpu.VMEM((CH,), jnp.int32)] * 2,
            [pltpu.VMEM((CH,), jnp.int32)] * 2,
            [pltpu.VMEM((CH, FW), jnp.float32)] * 2,
            [pltpu.VMEM((CH, 16), jnp.float32)] * 2,
            [pltpu.VMEM((CH, FW), jnp.float32)] * 2,
            pltpu.VMEM_SHARED((_NACC, FW), jnp.float32),
            [pltpu.SemaphoreType.DMA] * 2,
            [pltpu.SemaphoreType.DMA] * 2,
            [pltpu.SemaphoreType.DMA] * 2,
        ],
    )(hx, adst, src, dst)


def kernel(x, edge_index, W1, att_src1, att_dst1, bias1, W2, att_src2, att_dst2, bias2):
    n = x.shape[0]
    loop = jnp.arange(n, dtype=edge_index.dtype)
    ei = jnp.concatenate([edge_index, jnp.stack([loop, loop])], axis=1)
    e_tot = ei.shape[1]
    per = _CHUNK * _NCORE * _NSUB
    n_chunks = -(-e_tot // per)
    e_pad = n_chunks * per
    src = jnp.concatenate([ei[0], jnp.zeros((e_pad - e_tot,), jnp.int32)])
    dst = jnp.concatenate([ei[1], jnp.zeros((e_pad - e_tot,), jnp.int32)])

    heads, hid = att_src1.shape
    eye = jnp.eye(heads, dtype=jnp.float32)
    asrc_blk = (eye[:, None, :] * att_src1[:, :, None]).reshape(heads * hid, heads)
    adst_blk = (eye[:, None, :] * att_dst1[:, :, None]).reshape(heads * hid, heads)
    wext1 = jnp.concatenate(
        [W1, W1 @ asrc_blk, jnp.zeros((W1.shape[0], 8), jnp.float32)], axis=1)
    wdst1 = jnp.concatenate(
        [W1 @ adst_blk, jnp.zeros((W1.shape[0], 8), jnp.float32)], axis=1)
    rmat = (eye[:, None, :] * jnp.ones((heads, hid, 1), jnp.float32)).reshape(
        heads * hid, heads).T
    w2ext = jnp.concatenate(
        [W2, W2 @ att_src2.T, jnp.zeros((W2.shape[0], 15), jnp.float32)], axis=1)
    w2dst = jnp.concatenate(
        [W2 @ att_dst2.T, jnp.zeros((W2.shape[0], 15), jnp.float32)], axis=1)

    hx, ad1 = _tc1(x, wext1, wdst1)
    p0, p1 = _sc_edge(128, 8, 144, n_chunks, e_tot, hx, ad1, src, dst)
    hx2, ad2 = _tc2(p0, p1, bias1.reshape(1, 128), rmat, w2ext, w2dst)
    q0, q1 = _sc_edge(64, 1, 80, n_chunks, e_tot, hx2, ad2, src, dst)
    return _tc3(q0, q1, bias2.reshape(1, 64))
